# baseline (speedup 1.0000x reference)
import functools

import jax
import jax.numpy as jnp
from jax import lax
from jax.experimental import pallas as pl
from jax.experimental.pallas import tpu as pltpu
from jax.experimental.pallas import tpu_sc as plsc

N = 10000
E = 320000
IN = 128
HID = 16
HEADS = 8
OUT = 64

NP = 10240
BM = 256
GRID = NP // BM

NC = 2
NS = 16
L = 16
B = 128
CPT = 80
NCHUNK_P = NC * NS * CPT
EP = NCHUNK_P * B
ROWS = NP // NS

_mesh = plsc.VectorSubcoreMesh(
    core_axis_name="c", subcore_axis_name="s", num_cores=NC, num_subcores=NS)
_scparams = pltpu.CompilerParams(
    use_tc_tiling_on_sc=False, needs_layout_passes=False)


def _bcast(v, idx):
    dn = lax.GatherDimensionNumbers(
        offset_dims=(), collapsed_slice_dims=(0,), start_index_map=(0,))
    return lax.gather(v, idx[:, None], dn, (1,),
                      mode=lax.GatherScatterMode.PROMISE_IN_BOUNDS)


def _tc1_body(x_ref, w_ref, a_ref, h_ref, sa_ref, sb_ref, mx_ref):
    h = jnp.dot(x_ref[...], w_ref[...], preferred_element_type=jnp.float32)
    h_ref[...] = h
    s = jnp.dot(h, a_ref[...], preferred_element_type=jnp.float32)
    sa_ref[...] = s
    sb_ref[...] = jnp.concatenate([s[:, 8:], s[:, :8]], axis=1)
    m = jnp.max(s, axis=0, keepdims=True)

    @pl.when(pl.program_id(0) == 0)
    def _():
        mx_ref[...] = m

    @pl.when(pl.program_id(0) != 0)
    def _():
        mx_ref[...] = jnp.maximum(mx_ref[...], m)


def _tc1(x_pad, W1, A1):
    return pl.pallas_call(
        _tc1_body,
        grid=(GRID,),
        in_specs=[
            pl.BlockSpec((BM, IN), lambda i: (i, 0)),
            pl.BlockSpec((IN, HEADS * HID), lambda i: (0, 0)),
            pl.BlockSpec((HEADS * HID, 16), lambda i: (0, 0)),
        ],
        out_specs=[
            pl.BlockSpec((BM, HEADS * HID), lambda i: (i, 0)),
            pl.BlockSpec((BM, 16), lambda i: (i, 0)),
            pl.BlockSpec((BM, 16), lambda i: (i, 0)),
            pl.BlockSpec((1, 16), lambda i: (0, 0)),
        ],
        out_shape=[
            jax.ShapeDtypeStruct((NP, HEADS * HID), jnp.float32),
            jax.ShapeDtypeStruct((NP, 16), jnp.float32),
            jax.ShapeDtypeStruct((NP, 16), jnp.float32),
            jax.ShapeDtypeStruct((1, 16), jnp.float32),
        ],
    )(x_pad, W1, A1)


def _tc2_body(p0_ref, p1_ref, b_ref, w_ref, a_ref, h_ref, sa_ref, mx_ref):
    t = jnp.maximum(p0_ref[...] + p1_ref[...] + b_ref[...], 0.0)
    h = jnp.dot(t, w_ref[...], preferred_element_type=jnp.float32)
    h_ref[...] = h
    s = jnp.dot(h, a_ref[...], preferred_element_type=jnp.float32)
    sa_ref[...] = s
    m = jnp.max(s, axis=0, keepdims=True)

    @pl.when(pl.program_id(0) == 0)
    def _():
        mx_ref[...] = m

    @pl.when(pl.program_id(0) != 0)
    def _():
        mx_ref[...] = jnp.maximum(mx_ref[...], m)


def _tc2(p0, p1, b1r, W2, A2):
    return pl.pallas_call(
        _tc2_body,
        grid=(GRID,),
        in_specs=[
            pl.BlockSpec((BM, HEADS * HID), lambda i: (i, 0)),
            pl.BlockSpec((BM, HEADS * HID), lambda i: (i, 0)),
            pl.BlockSpec((1, HEADS * HID), lambda i: (0, 0)),
            pl.BlockSpec((HEADS * HID, OUT), lambda i: (0, 0)),
            pl.BlockSpec((OUT, 16), lambda i: (0, 0)),
        ],
        out_specs=[
            pl.BlockSpec((BM, OUT), lambda i: (i, 0)),
            pl.BlockSpec((BM, 16), lambda i: (i, 0)),
            pl.BlockSpec((1, 16), lambda i: (0, 0)),
        ],
        out_shape=[
            jax.ShapeDtypeStruct((NP, OUT), jnp.float32),
            jax.ShapeDtypeStruct((NP, 16), jnp.float32),
            jax.ShapeDtypeStruct((1, 16), jnp.float32),
        ],
    )(p0, p1, b1r, W2, A2)


def _tc3_body(q0_ref, q1_ref, b_ref, o_ref):
    o = q0_ref[...] + q1_ref[...] + b_ref[...]
    m = jnp.max(o, axis=1, keepdims=True)
    z = o - m
    lse = jnp.log(jnp.sum(jnp.exp(z), axis=1, keepdims=True))
    o_ref[...] = z - lse


def _tc3(q0, q1, b2r):
    return pl.pallas_call(
        _tc3_body,
        grid=(GRID,),
        in_specs=[
            pl.BlockSpec((BM, OUT), lambda i: (i, 0)),
            pl.BlockSpec((BM, OUT), lambda i: (i, 0)),
            pl.BlockSpec((1, OUT), lambda i: (0, 0)),
        ],
        out_specs=pl.BlockSpec((BM, OUT), lambda i: (i, 0)),
        out_shape=jax.ShapeDtypeStruct((NP, OUT), jnp.float32),
    )(q0, q1, b2r)


def _tcadd_body(a_ref, b_ref, o_ref):
    o_ref[...] = a_ref[...] + b_ref[...]


def _tc_add(a, b):
    rows, cols = a.shape
    bm = min(rows, BM)
    return pl.pallas_call(
        _tcadd_body,
        grid=(rows // bm,),
        in_specs=[
            pl.BlockSpec((bm, cols), lambda i: (i, 0)),
            pl.BlockSpec((bm, cols), lambda i: (i, 0)),
        ],
        out_specs=pl.BlockSpec((bm, cols), lambda i: (i, 0)),
        out_shape=jax.ShapeDtypeStruct((rows, cols), jnp.float32),
    )(a, b)


@functools.partial(
    pl.kernel,
    out_type=jax.ShapeDtypeStruct((NC, NP, 16), jnp.float32),
    mesh=_mesh,
    compiler_params=_scparams,
    scratch_types=[
        pltpu.VMEM((CPT, B), jnp.int32),
        pltpu.VMEM((CPT, B), jnp.int32),
        pltpu.VMEM((B, 16), jnp.float32),
        pltpu.VMEM((B, 16), jnp.float32),
        pltpu.VMEM((B, 16), jnp.float32),
        pltpu.VMEM((B, 16), jnp.float32),
        pltpu.VMEM((B, 16), jnp.float32),
        pltpu.VMEM((L,), jnp.float32),
        pltpu.VMEM_SHARED((NP, 16), jnp.float32),
        pltpu.SemaphoreType.DMA,
        pltpu.SemaphoreType.DMA,
        pltpu.SemaphoreType.DMA,
        pltpu.SemaphoreType.DMA,
    ],
)
def _sc1_pass1(src_hbm, dst_hbm, ta_hbm, tb_hbm, k_hbm, z_hbm, den_hbm,
               src_v, dst_v, sa0, sa1, sb0, sb1, ex_v, k_v, den_sp,
               sma0, sma1, smb0, smb1):
    c = lax.axis_index("c")
    s = lax.axis_index("s")
    wid = c * NS + s
    pltpu.sync_copy(k_hbm, k_v)
    kv = k_v[...]
    pltpu.sync_copy(z_hbm.at[pl.ds(s * ROWS, ROWS), :],
                    den_sp.at[pl.ds(s * ROWS, ROWS), :])
    pltpu.sync_copy(src_hbm.at[pl.ds(wid * CPT, CPT), :], src_v)
    pltpu.sync_copy(dst_hbm.at[pl.ds(wid * CPT, CPT), :], dst_v)
    plsc.subcore_barrier()

    bufs = ((sa0, sb0, sma0, smb0), (sa1, sb1, sma1, smb1))
    for par in range(2):
        sa, sb, sma, smb = bufs[par]
        pltpu.async_copy(ta_hbm.at[src_v.at[par]], sa, sma)
        pltpu.async_copy(tb_hbm.at[dst_v.at[par]], sb, smb)

    def pair(jj, carry):
        for par in range(2):
            j = jj * 2 + par
            sa, sb, sma, smb = bufs[par]
            pltpu.make_async_copy(ta_hbm.at[src_v.at[j]], sa, sma).wait()
            pltpu.make_async_copy(tb_hbm.at[dst_v.at[j]], sb, smb).wait()

            @plsc.parallel_loop(0, B, 1, unroll=4)
            def _(i):
                e = sa[i, :] + sb[i, :]
                e = jnp.maximum(e, 0.2 * e)
                ex_v[i, :] = jnp.exp(e - kv)

            pltpu.sync_copy(ex_v, den_sp.at[dst_v.at[j]], add=True)

            @pl.when(j + 2 < CPT)
            def _():
                pltpu.async_copy(ta_hbm.at[src_v.at[j + 2]], sa, sma)
                pltpu.async_copy(tb_hbm.at[dst_v.at[j + 2]], sb, smb)

        return carry

    lax.fori_loop(0, CPT // 2, pair, 0)
    plsc.subcore_barrier()
    pltpu.sync_copy(den_sp.at[pl.ds(s * ROWS, ROWS), :],
                    den_hbm.at[c, pl.ds(s * ROWS, ROWS), :])


@functools.partial(
    pl.kernel,
    out_type=jax.ShapeDtypeStruct((NC, NP, HEADS * HID), jnp.float32),
    mesh=_mesh,
    compiler_params=_scparams,
    scratch_types=[
        pltpu.VMEM((1, B), jnp.int32),
        pltpu.VMEM((1, B), jnp.int32),
        pltpu.VMEM((1, B), jnp.int32),
        pltpu.VMEM((1, B), jnp.int32),
        pltpu.VMEM((B, 16), jnp.float32),
        pltpu.VMEM((B, 16), jnp.float32),
        pltpu.VMEM((B, 16), jnp.float32),
        pltpu.VMEM((B, 16), jnp.float32),
        pltpu.VMEM((B, 16), jnp.float32),
        pltpu.VMEM((B, 16), jnp.float32),
        pltpu.VMEM((B, HEADS * HID), jnp.float32),
        pltpu.VMEM((B, HEADS * HID), jnp.float32),
        pltpu.VMEM((L,), jnp.float32),
        pltpu.VMEM_SHARED((NP, HEADS * HID), jnp.float32),
        pltpu.SemaphoreType.DMA,
        pltpu.SemaphoreType.DMA,
        pltpu.SemaphoreType.DMA,
        pltpu.SemaphoreType.DMA,
        pltpu.SemaphoreType.DMA,
        pltpu.SemaphoreType.DMA,
        pltpu.SemaphoreType.DMA,
        pltpu.SemaphoreType.DMA,
        pltpu.SemaphoreType.DMA,
        pltpu.SemaphoreType.DMA,
        pltpu.SemaphoreType.DMA,
        pltpu.SemaphoreType.DMA,
    ],
)
def _sc1_pass2(src_hbm, dst_hbm, ta_hbm, tb_hbm, k_hbm, den_hbm, h_hbm,
               z_hbm, out_hbm,
               s0, s1, d0, d1, sa0, sa1, sb0, sb1, dn0, dn1, h0, h1, k_v,
               out_sp,
               si0, si1, di0, di1, sma0, sma1, smb0, smb1, smd0, smd1,
               smh0, smh1):
    c = lax.axis_index("c")
    s = lax.axis_index("s")
    wid = c * NS + s
    base = wid * CPT
    pltpu.sync_copy(k_hbm, k_v)
    kv = k_v[...]
    mod8 = lax.iota(jnp.int32, L) % HEADS
    pltpu.sync_copy(z_hbm.at[pl.ds(s * ROWS, ROWS), :],
                    out_sp.at[pl.ds(s * ROWS, ROWS), :])
    plsc.subcore_barrier()

    ibufs = ((s0, d0, si0, di0), (s1, d1, si1, di1))
    rbufs = ((sa0, sb0, dn0, h0, sma0, smb0, smd0, smh0),
             (sa1, sb1, dn1, h1, sma1, smb1, smd1, smh1))

    def idx_copy(j, par):
        sv, dv, sis, dis = ibufs[par]
        pltpu.async_copy(src_hbm.at[pl.ds(base + j, 1), :], sv, sis)
        pltpu.async_copy(dst_hbm.at[pl.ds(base + j, 1), :], dv, dis)

    def idx_wait(j, par):
        sv, dv, sis, dis = ibufs[par]
        pltpu.make_async_copy(
            src_hbm.at[pl.ds(base + j, 1), :], sv, sis).wait()
        pltpu.make_async_copy(
            dst_hbm.at[pl.ds(base + j, 1), :], dv, dis).wait()

    def row_issue(par):
        sv, dv, _, _ = ibufs[par]
        sa, sb, dn, h, sma, smb, smd, smh = rbufs[par]
        pltpu.async_copy(ta_hbm.at[sv.at[0]], sa, sma)
        pltpu.async_copy(tb_hbm.at[dv.at[0]], sb, smb)
        pltpu.async_copy(den_hbm.at[dv.at[0]], dn, smd)
        pltpu.async_copy(h_hbm.at[sv.at[0]], h, smh)

    def row_wait(par):
        sv, dv, _, _ = ibufs[par]
        sa, sb, dn, h, sma, smb, smd, smh = rbufs[par]
        pltpu.make_async_copy(ta_hbm.at[sv.at[0]], sa, sma).wait()
        pltpu.make_async_copy(tb_hbm.at[dv.at[0]], sb, smb).wait()
        pltpu.make_async_copy(den_hbm.at[dv.at[0]], dn, smd).wait()
        pltpu.make_async_copy(h_hbm.at[sv.at[0]], h, smh).wait()

    idx_copy(0, 0)
    idx_wait(0, 0)
    row_issue(0)
    idx_copy(1, 1)

    def pair(jj, carry):
        for par in range(2):
            j = jj * 2 + par

            @pl.when(j + 1 < CPT)
            def _():
                idx_wait(j + 1, par ^ 1)
                row_issue(par ^ 1)

            row_wait(par)
            sa, sb, dn, h = rbufs[par][:4]

            @plsc.parallel_loop(0, B, 1, unroll=2)
            def _(i):
                e = sa[i, :] + sb[i, :]
                e = jnp.maximum(e, 0.2 * e)
                ex = jnp.exp(e - kv)
                al = ex / (dn[i, :] + 1e-16)
                av = _bcast(al, mod8)
                for k in range(HEADS):
                    h[i, pl.ds(k * HID, HID)] = (
                        h[i, pl.ds(k * HID, HID)] * av)

            dv = ibufs[par][1]
            pltpu.sync_copy(h, out_sp.at[dv.at[0]], add=True)

            @pl.when(j + 2 < CPT)
            def _():
                idx_copy(j + 2, par)

        return carry

    lax.fori_loop(0, CPT // 2, pair, 0)
    plsc.subcore_barrier()
    pltpu.sync_copy(out_sp.at[pl.ds(s * ROWS, ROWS), :],
                    out_hbm.at[c, pl.ds(s * ROWS, ROWS), :])


@functools.partial(
    pl.kernel,
    out_type=jax.ShapeDtypeStruct((NC, NP), jnp.float32),
    mesh=_mesh,
    compiler_params=_scparams,
    scratch_types=[
        pltpu.VMEM((CPT, B), jnp.int32),
        pltpu.VMEM((CPT, B), jnp.int32),
        pltpu.VMEM((B,), jnp.float32),
        pltpu.VMEM((NP,), jnp.float32),
        pltpu.VMEM((NP,), jnp.float32),
        pltpu.VMEM((L,), jnp.float32),
        pltpu.VMEM_SHARED((NP,), jnp.float32),
    ],
)
def _sc2_pass1(src_hbm, dst_hbm, as_hbm, ad_hbm, k_hbm, z_hbm, den_hbm,
               src_v, dst_v, ex_v, as_v, ad_v, k_v, den_sp):
    c = lax.axis_index("c")
    s = lax.axis_index("s")
    wid = c * NS + s
    pltpu.sync_copy(k_hbm, k_v)
    kv = k_v[...]
    pltpu.sync_copy(as_hbm, as_v)
    pltpu.sync_copy(ad_hbm, ad_v)
    pltpu.sync_copy(z_hbm.at[pl.ds(s * ROWS, ROWS)],
                    den_sp.at[pl.ds(s * ROWS, ROWS)])
    pltpu.sync_copy(src_hbm.at[pl.ds(wid * CPT, CPT), :], src_v)
    pltpu.sync_copy(dst_hbm.at[pl.ds(wid * CPT, CPT), :], dst_v)
    plsc.subcore_barrier()

    def chunk(j, carry):

        @plsc.parallel_loop(0, B // L, 1, unroll=4)
        def _(j2):
            sl = pl.ds(j2 * L, L)
            sv = plsc.load_gather(as_v, [src_v[j, sl]])
            dv = plsc.load_gather(ad_v, [dst_v[j, sl]])
            e = sv + dv
            e = jnp.maximum(e, 0.2 * e)
            ex_v[sl] = jnp.exp(e - kv)

        pltpu.sync_copy(ex_v, den_sp.at[dst_v.at[j]], add=True)
        return carry

    lax.fori_loop(0, CPT, chunk, 0)
    plsc.subcore_barrier()
    pltpu.sync_copy(den_sp.at[pl.ds(s * ROWS, ROWS)],
                    den_hbm.at[c, pl.ds(s * ROWS, ROWS)])


@functools.partial(
    pl.kernel,
    out_type=jax.ShapeDtypeStruct((NC, NP, OUT), jnp.float32),
    mesh=_mesh,
    compiler_params=_scparams,
    scratch_types=[
        pltpu.VMEM((CPT, B), jnp.int32),
        pltpu.VMEM((CPT, B), jnp.int32),
        pltpu.VMEM((B,), jnp.float32),
        pltpu.VMEM((NP,), jnp.float32),
        pltpu.VMEM((NP,), jnp.float32),
        pltpu.VMEM((NP,), jnp.float32),
        pltpu.VMEM((B, OUT), jnp.float32),
        pltpu.VMEM((B, OUT), jnp.float32),
        pltpu.VMEM((B, OUT), jnp.float32),
        pltpu.VMEM((L,), jnp.float32),
        pltpu.VMEM_SHARED((NP, OUT), jnp.float32),
        pltpu.SemaphoreType.DMA,
        pltpu.SemaphoreType.DMA,
    ],
)
def _sc2_pass2(src_hbm, dst_hbm, as_hbm, ad_hbm, k_hbm, den_hbm, h_hbm,
               z_hbm, out_hbm,
               src_v, dst_v, al_v, as_v, ad_v, dn_v, h0, h1, msg_v, k_v,
               out_sp, smh0, smh1):
    c = lax.axis_index("c")
    s = lax.axis_index("s")
    wid = c * NS + s
    pltpu.sync_copy(k_hbm, k_v)
    kv = k_v[...]
    pltpu.sync_copy(as_hbm, as_v)
    pltpu.sync_copy(ad_hbm, ad_v)
    pltpu.sync_copy(den_hbm, dn_v)
    pltpu.sync_copy(z_hbm.at[pl.ds(s * ROWS, ROWS), :],
                    out_sp.at[pl.ds(s * ROWS, ROWS), :])
    pltpu.sync_copy(src_hbm.at[pl.ds(wid * CPT, CPT), :], src_v)
    pltpu.sync_copy(dst_hbm.at[pl.ds(wid * CPT, CPT), :], dst_v)
    plsc.subcore_barrier()

    bufs = ((h0, smh0), (h1, smh1))
    for par in range(2):
        h, smh = bufs[par]
        pltpu.async_copy(h_hbm.at[src_v.at[par]], h, smh)

    def pair(jj, carry):
        for par in range(2):
            j = jj * 2 + par
            h, smh = bufs[par]

            @plsc.parallel_loop(0, B // L, 1, unroll=4)
            def _(j2):
                sl = pl.ds(j2 * L, L)
                sv = plsc.load_gather(as_v, [src_v[j, sl]])
                d = dst_v[j, sl]
                dv = plsc.load_gather(ad_v, [d])
                e = sv + dv
                e = jnp.maximum(e, 0.2 * e)
                ex = jnp.exp(e - kv)
                den = plsc.load_gather(dn_v, [d])
                al_v[sl] = ex / (den + 1e-16)

            pltpu.make_async_copy(h_hbm.at[src_v.at[j]], h, smh).wait()

            @plsc.parallel_loop(0, B // L, 1, unroll=1)
            def _(g):
                avall = al_v[pl.ds(g * L, L)]
                for m in range(L):
                    av = _bcast(avall, jnp.full((L,), m, jnp.int32))
                    i = g * L + m
                    for k in range(OUT // HID):
                        msg_v[i, pl.ds(k * HID, HID)] = (
                            h[i, pl.ds(k * HID, HID)] * av)

            pltpu.sync_copy(msg_v, out_sp.at[dst_v.at[j]], add=True)

            @pl.when(j + 2 < CPT)
            def _():
                pltpu.async_copy(h_hbm.at[src_v.at[j + 2]], h, smh)

        return carry

    lax.fori_loop(0, CPT // 2, pair, 0)
    plsc.subcore_barrier()
    pltpu.sync_copy(out_sp.at[pl.ds(s * ROWS, ROWS), :],
                    out_hbm.at[c, pl.ds(s * ROWS, ROWS), :])


def kernel(x, edge_index, W1, a_src1, a_dst1, b1, W2, a_src2, a_dst2, b2):
    x_pad = jnp.pad(x, ((0, NP - N), (0, 0)))
    src = edge_index[0].astype(jnp.int32)
    dst = edge_index[1].astype(jnp.int32)
    pad = jnp.full((EP - E,), NP - 1, jnp.int32)
    src_p = jnp.concatenate([src, pad]).reshape(NCHUNK_P, B)
    dst_p = jnp.concatenate([dst, pad]).reshape(NCHUNK_P, B)

    cols = jnp.arange(HEADS * HID)
    orig = (cols % HEADS) * HID + cols // HEADS
    W1p = W1[:, orig]
    b1p = b1[orig]
    W2p = W2[orig, :]

    eye = jnp.repeat(jnp.eye(HEADS, dtype=jnp.float32), HID, axis=0)
    A_src = eye * a_src1.reshape(-1)[:, None]
    A_dst = eye * a_dst1.reshape(-1)[:, None]
    A1 = jnp.concatenate([A_src, A_dst], axis=1)[orig, :]

    h1, ta1, tb1, mx1 = _tc1(x_pad, W1p, A1)
    k1 = mx1[0, :8] + mx1[0, 8:]
    k1 = jnp.maximum(k1, 0.2 * k1)
    k16_1 = jnp.concatenate([k1, k1])

    z16 = jnp.zeros((NP, 16), jnp.float32)
    z128 = jnp.zeros((NP, HEADS * HID), jnp.float32)
    den1p = _sc1_pass1(src_p, dst_p, ta1, tb1, k16_1, z16)
    den1 = _tc_add(den1p[0], den1p[1])
    out1p = _sc1_pass2(src_p, dst_p, ta1, tb1, k16_1, den1, h1, z128)

    A2 = jnp.zeros((OUT, 16), jnp.float32)
    A2 = A2.at[:, 0].set(a_src2[0]).at[:, 1].set(a_dst2[0])
    h2, sa2, mx2 = _tc2(out1p[0], out1p[1], b1p.reshape(1, -1), W2p, A2)
    k2 = mx2[0, 0] + mx2[0, 1]
    k2 = jnp.maximum(k2, 0.2 * k2)
    k16_2 = jnp.full((L,), k2, jnp.float32)
    as2 = sa2[:, 0]
    ad2 = sa2[:, 1]

    z1d = jnp.zeros((NP,), jnp.float32)
    z64 = jnp.zeros((NP, OUT), jnp.float32)
    den2p = _sc2_pass1(src_p, dst_p, as2, ad2, k16_2, z1d)
    den2 = _tc_add(den2p[0].reshape(80, 128),
                   den2p[1].reshape(80, 128)).reshape(NP)
    out2p = _sc2_pass2(src_p, dst_p, as2, ad2, k16_2, den2, h2, z64)

    o = _tc3(out2p[0], out2p[1], b2.reshape(1, -1))
    return o[:N]


# speedup vs baseline: 1.8398x; 1.8398x over previous
import functools

import jax
import jax.numpy as jnp
from jax import lax
from jax.experimental import pallas as pl
from jax.experimental.pallas import tpu as pltpu
from jax.experimental.pallas import tpu_sc as plsc

N = 10000
E = 320000
IN = 128
HID = 16
HEADS = 8
OUT = 64

NP = 10240
BM = 256
GRID = NP // BM

NC = 2
NS = 16
L = 16
B = 128
CPT = 80
NCHUNK_P = NC * NS * CPT
EP = NCHUNK_P * B
ROWS = NP // NS

_mesh = plsc.VectorSubcoreMesh(
    core_axis_name="c", subcore_axis_name="s", num_cores=NC, num_subcores=NS)
_scparams = pltpu.CompilerParams(
    use_tc_tiling_on_sc=False, needs_layout_passes=False)


def _bcast(v, idx):
    dn = lax.GatherDimensionNumbers(
        offset_dims=(), collapsed_slice_dims=(0,), start_index_map=(0,))
    return lax.gather(v, idx[:, None], dn, (1,),
                      mode=lax.GatherScatterMode.PROMISE_IN_BOUNDS)


def _tc1_body(x_ref, w_ref, a_ref, h_ref, sa_ref, sb_ref, mx_ref):
    h = jnp.dot(x_ref[...], w_ref[...], preferred_element_type=jnp.float32)
    h_ref[...] = h
    s = jnp.dot(h, a_ref[...], preferred_element_type=jnp.float32)
    sa_ref[...] = s
    sb_ref[...] = jnp.concatenate([s[:, 8:], s[:, :8]], axis=1)
    m = jnp.max(s, axis=0, keepdims=True)

    @pl.when(pl.program_id(0) == 0)
    def _():
        mx_ref[...] = m

    @pl.when(pl.program_id(0) != 0)
    def _():
        mx_ref[...] = jnp.maximum(mx_ref[...], m)


def _tc1(x_pad, W1, A1):
    return pl.pallas_call(
        _tc1_body,
        grid=(GRID,),
        in_specs=[
            pl.BlockSpec((BM, IN), lambda i: (i, 0)),
            pl.BlockSpec((IN, HEADS * HID), lambda i: (0, 0)),
            pl.BlockSpec((HEADS * HID, 16), lambda i: (0, 0)),
        ],
        out_specs=[
            pl.BlockSpec((BM, HEADS * HID), lambda i: (i, 0)),
            pl.BlockSpec((BM, 16), lambda i: (i, 0)),
            pl.BlockSpec((BM, 16), lambda i: (i, 0)),
            pl.BlockSpec((1, 16), lambda i: (0, 0)),
        ],
        out_shape=[
            jax.ShapeDtypeStruct((NP, HEADS * HID), jnp.float32),
            jax.ShapeDtypeStruct((NP, 16), jnp.float32),
            jax.ShapeDtypeStruct((NP, 16), jnp.float32),
            jax.ShapeDtypeStruct((1, 16), jnp.float32),
        ],
    )(x_pad, W1, A1)


def _tc2_body(p0_ref, p1_ref, b_ref, w_ref, a_ref, h_ref, sa_ref, mx_ref):
    t = jnp.maximum(p0_ref[...] + p1_ref[...] + b_ref[...], 0.0)
    h = jnp.dot(t, w_ref[...], preferred_element_type=jnp.float32)
    h_ref[...] = h
    s = jnp.dot(h, a_ref[...], preferred_element_type=jnp.float32)
    sa_ref[...] = s
    m = jnp.max(s, axis=0, keepdims=True)

    @pl.when(pl.program_id(0) == 0)
    def _():
        mx_ref[...] = m

    @pl.when(pl.program_id(0) != 0)
    def _():
        mx_ref[...] = jnp.maximum(mx_ref[...], m)


def _tc2(p0, p1, b1r, W2, A2):
    return pl.pallas_call(
        _tc2_body,
        grid=(GRID,),
        in_specs=[
            pl.BlockSpec((BM, HEADS * HID), lambda i: (i, 0)),
            pl.BlockSpec((BM, HEADS * HID), lambda i: (i, 0)),
            pl.BlockSpec((1, HEADS * HID), lambda i: (0, 0)),
            pl.BlockSpec((HEADS * HID, OUT), lambda i: (0, 0)),
            pl.BlockSpec((OUT, 16), lambda i: (0, 0)),
        ],
        out_specs=[
            pl.BlockSpec((BM, OUT), lambda i: (i, 0)),
            pl.BlockSpec((BM, 16), lambda i: (i, 0)),
            pl.BlockSpec((1, 16), lambda i: (0, 0)),
        ],
        out_shape=[
            jax.ShapeDtypeStruct((NP, OUT), jnp.float32),
            jax.ShapeDtypeStruct((NP, 16), jnp.float32),
            jax.ShapeDtypeStruct((1, 16), jnp.float32),
        ],
    )(p0, p1, b1r, W2, A2)


def _tc3_body(q0_ref, q1_ref, b_ref, o_ref):
    o = q0_ref[...] + q1_ref[...] + b_ref[...]
    m = jnp.max(o, axis=1, keepdims=True)
    z = o - m
    lse = jnp.log(jnp.sum(jnp.exp(z), axis=1, keepdims=True))
    o_ref[...] = z - lse


def _tc3(q0, q1, b2r):
    return pl.pallas_call(
        _tc3_body,
        grid=(GRID,),
        in_specs=[
            pl.BlockSpec((BM, OUT), lambda i: (i, 0)),
            pl.BlockSpec((BM, OUT), lambda i: (i, 0)),
            pl.BlockSpec((1, OUT), lambda i: (0, 0)),
        ],
        out_specs=pl.BlockSpec((BM, OUT), lambda i: (i, 0)),
        out_shape=jax.ShapeDtypeStruct((NP, OUT), jnp.float32),
    )(q0, q1, b2r)


def _tcadd_body(a_ref, b_ref, o_ref):
    o_ref[...] = a_ref[...] + b_ref[...]


def _tc_add(a, b):
    rows, cols = a.shape
    bm = min(rows, BM)
    return pl.pallas_call(
        _tcadd_body,
        grid=(rows // bm,),
        in_specs=[
            pl.BlockSpec((bm, cols), lambda i: (i, 0)),
            pl.BlockSpec((bm, cols), lambda i: (i, 0)),
        ],
        out_specs=pl.BlockSpec((bm, cols), lambda i: (i, 0)),
        out_shape=jax.ShapeDtypeStruct((rows, cols), jnp.float32),
    )(a, b)


@functools.partial(
    pl.kernel,
    out_type=jax.ShapeDtypeStruct((NC, NP, 16), jnp.float32),
    mesh=_mesh,
    compiler_params=_scparams,
    scratch_types=[
        pltpu.VMEM((CPT, B), jnp.int32),
        pltpu.VMEM((CPT, B), jnp.int32),
        pltpu.VMEM((B, 16), jnp.float32),
        pltpu.VMEM((B, 16), jnp.float32),
        pltpu.VMEM((B, 16), jnp.float32),
        pltpu.VMEM((B, 16), jnp.float32),
        pltpu.VMEM((B, 16), jnp.float32),
        pltpu.VMEM((L,), jnp.float32),
        pltpu.VMEM_SHARED((NP, 16), jnp.float32),
        pltpu.SemaphoreType.DMA,
        pltpu.SemaphoreType.DMA,
        pltpu.SemaphoreType.DMA,
        pltpu.SemaphoreType.DMA,
    ],
)
def _sc1_pass1(src_hbm, dst_hbm, ta_hbm, tb_hbm, k_hbm, z_hbm, den_hbm,
               src_v, dst_v, sa0, sa1, sb0, sb1, ex_v, k_v, den_sp,
               sma0, sma1, smb0, smb1):
    c = lax.axis_index("c")
    s = lax.axis_index("s")
    wid = c * NS + s
    pltpu.sync_copy(k_hbm, k_v)
    kv = k_v[...]
    pltpu.sync_copy(z_hbm.at[pl.ds(s * ROWS, ROWS), :],
                    den_sp.at[pl.ds(s * ROWS, ROWS), :])
    pltpu.sync_copy(src_hbm.at[pl.ds(wid * CPT, CPT), :], src_v)
    pltpu.sync_copy(dst_hbm.at[pl.ds(wid * CPT, CPT), :], dst_v)
    plsc.subcore_barrier()

    bufs = ((sa0, sb0, sma0, smb0), (sa1, sb1, sma1, smb1))
    for par in range(2):
        sa, sb, sma, smb = bufs[par]
        pltpu.async_copy(ta_hbm.at[src_v.at[par]], sa, sma)
        pltpu.async_copy(tb_hbm.at[dst_v.at[par]], sb, smb)

    def pair(jj, carry):
        for par in range(2):
            j = jj * 2 + par
            sa, sb, sma, smb = bufs[par]
            pltpu.make_async_copy(ta_hbm.at[src_v.at[j]], sa, sma).wait()
            pltpu.make_async_copy(tb_hbm.at[dst_v.at[j]], sb, smb).wait()

            @plsc.parallel_loop(0, B, 1, unroll=4)
            def _(i):
                e = sa[i, :] + sb[i, :]
                e = jnp.maximum(e, 0.2 * e)
                ex_v[i, :] = jnp.exp(e - kv)

            pltpu.sync_copy(ex_v, den_sp.at[dst_v.at[j]], add=True)

            @pl.when(j + 2 < CPT)
            def _():
                pltpu.async_copy(ta_hbm.at[src_v.at[j + 2]], sa, sma)
                pltpu.async_copy(tb_hbm.at[dst_v.at[j + 2]], sb, smb)

        return carry

    lax.fori_loop(0, CPT // 2, pair, 0)
    plsc.subcore_barrier()
    pltpu.sync_copy(den_sp.at[pl.ds(s * ROWS, ROWS), :],
                    den_hbm.at[c, pl.ds(s * ROWS, ROWS), :])


@functools.partial(
    pl.kernel,
    out_type=jax.ShapeDtypeStruct((NC, NP, HEADS * HID), jnp.float32),
    mesh=_mesh,
    compiler_params=_scparams,
    scratch_types=[
        pltpu.VMEM((1, B), jnp.int32),
        pltpu.VMEM((1, B), jnp.int32),
        pltpu.VMEM((1, B), jnp.int32),
        pltpu.VMEM((1, B), jnp.int32),
        pltpu.VMEM((B, 16), jnp.float32),
        pltpu.VMEM((B, 16), jnp.float32),
        pltpu.VMEM((B, 16), jnp.float32),
        pltpu.VMEM((B, 16), jnp.float32),
        pltpu.VMEM((B, 16), jnp.float32),
        pltpu.VMEM((B, 16), jnp.float32),
        pltpu.VMEM((B, HEADS * HID), jnp.float32),
        pltpu.VMEM((B, HEADS * HID), jnp.float32),
        pltpu.VMEM((L,), jnp.float32),
        pltpu.VMEM_SHARED((NP, HEADS * HID), jnp.float32),
        pltpu.SemaphoreType.DMA,
        pltpu.SemaphoreType.DMA,
        pltpu.SemaphoreType.DMA,
        pltpu.SemaphoreType.DMA,
        pltpu.SemaphoreType.DMA,
        pltpu.SemaphoreType.DMA,
        pltpu.SemaphoreType.DMA,
        pltpu.SemaphoreType.DMA,
        pltpu.SemaphoreType.DMA,
        pltpu.SemaphoreType.DMA,
        pltpu.SemaphoreType.DMA,
        pltpu.SemaphoreType.DMA,
    ],
)
def _sc1_pass2(src_hbm, dst_hbm, ta_hbm, tb_hbm, k_hbm, den_hbm, h_hbm,
               z_hbm, out_hbm,
               s0, s1, d0, d1, sa0, sa1, sb0, sb1, dn0, dn1, h0, h1, k_v,
               out_sp,
               si0, si1, di0, di1, sma0, sma1, smb0, smb1, smd0, smd1,
               smh0, smh1):
    c = lax.axis_index("c")
    s = lax.axis_index("s")
    wid = c * NS + s
    base = wid * CPT
    pltpu.sync_copy(k_hbm, k_v)
    kv = k_v[...]
    mod8 = lax.iota(jnp.int32, L) % HEADS
    pltpu.sync_copy(z_hbm.at[pl.ds(s * ROWS, ROWS), :],
                    out_sp.at[pl.ds(s * ROWS, ROWS), :])
    plsc.subcore_barrier()

    ibufs = ((s0, d0, si0, di0), (s1, d1, si1, di1))
    rbufs = ((sa0, sb0, dn0, h0, sma0, smb0, smd0, smh0),
             (sa1, sb1, dn1, h1, sma1, smb1, smd1, smh1))

    def idx_copy(j, par):
        sv, dv, sis, dis = ibufs[par]
        pltpu.async_copy(src_hbm.at[pl.ds(base + j, 1), :], sv, sis)
        pltpu.async_copy(dst_hbm.at[pl.ds(base + j, 1), :], dv, dis)

    def idx_wait(j, par):
        sv, dv, sis, dis = ibufs[par]
        pltpu.make_async_copy(
            src_hbm.at[pl.ds(base + j, 1), :], sv, sis).wait()
        pltpu.make_async_copy(
            dst_hbm.at[pl.ds(base + j, 1), :], dv, dis).wait()

    def row_issue(par):
        sv, dv, _, _ = ibufs[par]
        sa, sb, dn, h, sma, smb, smd, smh = rbufs[par]
        pltpu.async_copy(ta_hbm.at[sv.at[0]], sa, sma)
        pltpu.async_copy(tb_hbm.at[dv.at[0]], sb, smb)
        pltpu.async_copy(den_hbm.at[dv.at[0]], dn, smd)
        pltpu.async_copy(h_hbm.at[sv.at[0]], h, smh)

    def row_wait(par):
        sv, dv, _, _ = ibufs[par]
        sa, sb, dn, h, sma, smb, smd, smh = rbufs[par]
        pltpu.make_async_copy(ta_hbm.at[sv.at[0]], sa, sma).wait()
        pltpu.make_async_copy(tb_hbm.at[dv.at[0]], sb, smb).wait()
        pltpu.make_async_copy(den_hbm.at[dv.at[0]], dn, smd).wait()
        pltpu.make_async_copy(h_hbm.at[sv.at[0]], h, smh).wait()

    idx_copy(0, 0)
    idx_wait(0, 0)
    row_issue(0)
    idx_copy(1, 1)

    def pair(jj, carry):
        for par in range(2):
            j = jj * 2 + par

            @pl.when(j + 1 < CPT)
            def _():
                idx_wait(j + 1, par ^ 1)
                row_issue(par ^ 1)

            row_wait(par)
            sa, sb, dn, h = rbufs[par][:4]

            @plsc.parallel_loop(0, B, 1, unroll=2)
            def _(i):
                e = sa[i, :] + sb[i, :]
                e = jnp.maximum(e, 0.2 * e)
                ex = jnp.exp(e - kv)
                al = ex / (dn[i, :] + 1e-16)
                av = _bcast(al, mod8)
                for k in range(HEADS):
                    h[i, pl.ds(k * HID, HID)] = (
                        h[i, pl.ds(k * HID, HID)] * av)

            dv = ibufs[par][1]
            pltpu.sync_copy(h, out_sp.at[dv.at[0]], add=True)

            @pl.when(j + 2 < CPT)
            def _():
                idx_copy(j + 2, par)

        return carry

    lax.fori_loop(0, CPT // 2, pair, 0)
    plsc.subcore_barrier()
    pltpu.sync_copy(out_sp.at[pl.ds(s * ROWS, ROWS), :],
                    out_hbm.at[c, pl.ds(s * ROWS, ROWS), :])


@functools.partial(
    pl.kernel,
    out_type=jax.ShapeDtypeStruct((NC, NP), jnp.float32),
    mesh=_mesh,
    compiler_params=_scparams,
    scratch_types=[
        pltpu.VMEM((CPT, B), jnp.int32),
        pltpu.VMEM((CPT, B), jnp.int32),
        pltpu.VMEM((B,), jnp.float32),
        pltpu.VMEM((NP,), jnp.float32),
        pltpu.VMEM((NP,), jnp.float32),
        pltpu.VMEM((L,), jnp.float32),
        pltpu.VMEM_SHARED((NP,), jnp.float32),
    ],
)
def _sc2_pass1(src_hbm, dst_hbm, as_hbm, ad_hbm, k_hbm, z_hbm, den_hbm,
               src_v, dst_v, ex_v, as_v, ad_v, k_v, den_sp):
    c = lax.axis_index("c")
    s = lax.axis_index("s")
    wid = c * NS + s
    pltpu.sync_copy(k_hbm, k_v)
    kv = k_v[...]
    pltpu.sync_copy(as_hbm, as_v)
    pltpu.sync_copy(ad_hbm, ad_v)
    pltpu.sync_copy(z_hbm.at[pl.ds(s * ROWS, ROWS)],
                    den_sp.at[pl.ds(s * ROWS, ROWS)])
    pltpu.sync_copy(src_hbm.at[pl.ds(wid * CPT, CPT), :], src_v)
    pltpu.sync_copy(dst_hbm.at[pl.ds(wid * CPT, CPT), :], dst_v)
    plsc.subcore_barrier()

    def chunk(j, carry):

        @plsc.parallel_loop(0, B // L, 1, unroll=4)
        def _(j2):
            sl = pl.ds(j2 * L, L)
            sv = plsc.load_gather(as_v, [src_v[j, sl]])
            dv = plsc.load_gather(ad_v, [dst_v[j, sl]])
            e = sv + dv
            e = jnp.maximum(e, 0.2 * e)
            ex_v[sl] = jnp.exp(e - kv)

        pltpu.sync_copy(ex_v, den_sp.at[dst_v.at[j]], add=True)
        return carry

    lax.fori_loop(0, CPT, chunk, 0)
    plsc.subcore_barrier()
    pltpu.sync_copy(den_sp.at[pl.ds(s * ROWS, ROWS)],
                    den_hbm.at[c, pl.ds(s * ROWS, ROWS)])


@functools.partial(
    pl.kernel,
    out_type=jax.ShapeDtypeStruct((NC, NP, OUT), jnp.float32),
    mesh=_mesh,
    compiler_params=_scparams,
    scratch_types=[
        pltpu.VMEM((CPT, B), jnp.int32),
        pltpu.VMEM((CPT, B), jnp.int32),
        pltpu.VMEM((B,), jnp.float32),
        pltpu.VMEM((NP,), jnp.float32),
        pltpu.VMEM((NP,), jnp.float32),
        pltpu.VMEM((NP,), jnp.float32),
        pltpu.VMEM((B, OUT), jnp.float32),
        pltpu.VMEM((B, OUT), jnp.float32),
        pltpu.VMEM((B, OUT), jnp.float32),
        pltpu.VMEM((L,), jnp.float32),
        pltpu.VMEM_SHARED((NP, OUT), jnp.float32),
        pltpu.SemaphoreType.DMA,
        pltpu.SemaphoreType.DMA,
    ],
)
def _sc2_pass2(src_hbm, dst_hbm, as_hbm, ad_hbm, k_hbm, den_hbm, h_hbm,
               z_hbm, out_hbm,
               src_v, dst_v, al_v, as_v, ad_v, dn_v, h0, h1, msg_v, k_v,
               out_sp, smh0, smh1):
    c = lax.axis_index("c")
    s = lax.axis_index("s")
    wid = c * NS + s
    pltpu.sync_copy(k_hbm, k_v)
    kv = k_v[...]
    pltpu.sync_copy(as_hbm, as_v)
    pltpu.sync_copy(ad_hbm, ad_v)
    pltpu.sync_copy(den_hbm, dn_v)
    pltpu.sync_copy(z_hbm.at[pl.ds(s * ROWS, ROWS), :],
                    out_sp.at[pl.ds(s * ROWS, ROWS), :])
    pltpu.sync_copy(src_hbm.at[pl.ds(wid * CPT, CPT), :], src_v)
    pltpu.sync_copy(dst_hbm.at[pl.ds(wid * CPT, CPT), :], dst_v)
    plsc.subcore_barrier()

    bufs = ((h0, smh0), (h1, smh1))
    for par in range(2):
        h, smh = bufs[par]
        pltpu.async_copy(h_hbm.at[src_v.at[par]], h, smh)

    def pair(jj, carry):
        for par in range(2):
            j = jj * 2 + par
            h, smh = bufs[par]

            @plsc.parallel_loop(0, B // L, 1, unroll=4)
            def _(j2):
                sl = pl.ds(j2 * L, L)
                sv = plsc.load_gather(as_v, [src_v[j, sl]])
                d = dst_v[j, sl]
                dv = plsc.load_gather(ad_v, [d])
                e = sv + dv
                e = jnp.maximum(e, 0.2 * e)
                ex = jnp.exp(e - kv)
                den = plsc.load_gather(dn_v, [d])
                al_v[sl] = ex / (den + 1e-16)

            pltpu.make_async_copy(h_hbm.at[src_v.at[j]], h, smh).wait()

            @plsc.parallel_loop(0, B // L, 1, unroll=1)
            def _(g):
                avall = al_v[pl.ds(g * L, L)]
                for m in range(L):
                    av = _bcast(avall, jnp.full((L,), m, jnp.int32))
                    i = g * L + m
                    for k in range(OUT // HID):
                        msg_v[i, pl.ds(k * HID, HID)] = (
                            h[i, pl.ds(k * HID, HID)] * av)

            pltpu.sync_copy(msg_v, out_sp.at[dst_v.at[j]], add=True)

            @pl.when(j + 2 < CPT)
            def _():
                pltpu.async_copy(h_hbm.at[src_v.at[j + 2]], h, smh)

        return carry

    lax.fori_loop(0, CPT // 2, pair, 0)
    plsc.subcore_barrier()
    pltpu.sync_copy(out_sp.at[pl.ds(s * ROWS, ROWS), :],
                    out_hbm.at[c, pl.ds(s * ROWS, ROWS), :])


def kernel(x, edge_index, W1, a_src1, a_dst1, b1, W2, a_src2, a_dst2, b2):
    x_pad = jnp.pad(x, ((0, NP - N), (0, 0)))
    src = edge_index[0].astype(jnp.int32)
    dst = edge_index[1].astype(jnp.int32)
    pad = N + jnp.arange(EP - E, dtype=jnp.int32) % (NP - N)
    src_p = jnp.concatenate([src, pad]).reshape(NCHUNK_P, B)
    dst_p = jnp.concatenate([dst, pad]).reshape(NCHUNK_P, B)

    cols = jnp.arange(HEADS * HID)
    orig = (cols % HEADS) * HID + cols // HEADS
    W1p = W1[:, orig]
    b1p = b1[orig]
    W2p = W2[orig, :]

    eye = jnp.repeat(jnp.eye(HEADS, dtype=jnp.float32), HID, axis=0)
    A_src = eye * a_src1.reshape(-1)[:, None]
    A_dst = eye * a_dst1.reshape(-1)[:, None]
    A1 = jnp.concatenate([A_src, A_dst], axis=1)[orig, :]

    h1, ta1, tb1, mx1 = _tc1(x_pad, W1p, A1)
    k1 = mx1[0, :8] + mx1[0, 8:]
    k1 = jnp.maximum(k1, 0.2 * k1)
    k16_1 = jnp.concatenate([k1, k1])

    z16 = jnp.zeros((NP, 16), jnp.float32)
    z128 = jnp.zeros((NP, HEADS * HID), jnp.float32)
    den1p = _sc1_pass1(src_p, dst_p, ta1, tb1, k16_1, z16)
    den1 = _tc_add(den1p[0], den1p[1])
    out1p = _sc1_pass2(src_p, dst_p, ta1, tb1, k16_1, den1, h1, z128)

    A2 = jnp.zeros((OUT, 16), jnp.float32)
    A2 = A2.at[:, 0].set(a_src2[0]).at[:, 1].set(a_dst2[0])
    h2, sa2, mx2 = _tc2(out1p[0], out1p[1], b1p.reshape(1, -1), W2p, A2)
    k2 = mx2[0, 0] + mx2[0, 1]
    k2 = jnp.maximum(k2, 0.2 * k2)
    k16_2 = jnp.full((L,), k2, jnp.float32)
    as2 = sa2[:, 0]
    ad2 = sa2[:, 1]

    z1d = jnp.zeros((NP,), jnp.float32)
    z64 = jnp.zeros((NP, OUT), jnp.float32)
    den2p = _sc2_pass1(src_p, dst_p, as2, ad2, k16_2, z1d)
    den2 = _tc_add(den2p[0].reshape(80, 128),
                   den2p[1].reshape(80, 128)).reshape(NP)
    out2p = _sc2_pass2(src_p, dst_p, as2, ad2, k16_2, den2, h2, z64)

    o = _tc3(out2p[0], out2p[1], b2.reshape(1, -1))
    return o[:N]


# speedup vs baseline: 1.8416x; 1.0010x over previous
import functools

import jax
import jax.numpy as jnp
from jax import lax
from jax.experimental import pallas as pl
from jax.experimental.pallas import tpu as pltpu
from jax.experimental.pallas import tpu_sc as plsc

N = 10000
E = 320000
IN = 128
HID = 16
HEADS = 8
OUT = 64

NP = 10240
BM = 256
GRID = NP // BM

NC = 2
NS = 16
L = 16
B = 128
CPT = 80
NCHUNK_P = NC * NS * CPT
EP = NCHUNK_P * B
ROWS = NP // NS

_mesh = plsc.VectorSubcoreMesh(
    core_axis_name="c", subcore_axis_name="s", num_cores=NC, num_subcores=NS)
_scparams = pltpu.CompilerParams(
    use_tc_tiling_on_sc=False, needs_layout_passes=False)


def _bcast(v, idx):
    dn = lax.GatherDimensionNumbers(
        offset_dims=(), collapsed_slice_dims=(0,), start_index_map=(0,))
    return lax.gather(v, idx[:, None], dn, (1,),
                      mode=lax.GatherScatterMode.PROMISE_IN_BOUNDS)


def _tc1_body(x_ref, w_ref, a_ref, h_ref, sa_ref, sb_ref, mx_ref):
    h = jnp.dot(x_ref[...], w_ref[...], preferred_element_type=jnp.float32)
    h_ref[...] = h
    s = jnp.dot(h, a_ref[...], preferred_element_type=jnp.float32)
    sa_ref[...] = s
    sb_ref[...] = jnp.concatenate([s[:, 8:], s[:, :8]], axis=1)
    m = jnp.max(s, axis=0, keepdims=True)

    @pl.when(pl.program_id(0) == 0)
    def _():
        mx_ref[...] = m

    @pl.when(pl.program_id(0) != 0)
    def _():
        mx_ref[...] = jnp.maximum(mx_ref[...], m)


def _tc1(x_pad, W1, A1):
    return pl.pallas_call(
        _tc1_body,
        grid=(GRID,),
        in_specs=[
            pl.BlockSpec((BM, IN), lambda i: (i, 0)),
            pl.BlockSpec((IN, HEADS * HID), lambda i: (0, 0)),
            pl.BlockSpec((HEADS * HID, 16), lambda i: (0, 0)),
        ],
        out_specs=[
            pl.BlockSpec((BM, HEADS * HID), lambda i: (i, 0)),
            pl.BlockSpec((BM, 16), lambda i: (i, 0)),
            pl.BlockSpec((BM, 16), lambda i: (i, 0)),
            pl.BlockSpec((1, 16), lambda i: (0, 0)),
        ],
        out_shape=[
            jax.ShapeDtypeStruct((NP, HEADS * HID), jnp.float32),
            jax.ShapeDtypeStruct((NP, 16), jnp.float32),
            jax.ShapeDtypeStruct((NP, 16), jnp.float32),
            jax.ShapeDtypeStruct((1, 16), jnp.float32),
        ],
    )(x_pad, W1, A1)


def _tc2_body(p0_ref, p1_ref, b_ref, w_ref, a_ref, h_ref, sa_ref, mx_ref):
    t = jnp.maximum(p0_ref[...] + p1_ref[...] + b_ref[...], 0.0)
    h = jnp.dot(t, w_ref[...], preferred_element_type=jnp.float32)
    h_ref[...] = h
    s = jnp.dot(h, a_ref[...], preferred_element_type=jnp.float32)
    sa_ref[...] = s
    m = jnp.max(s, axis=0, keepdims=True)

    @pl.when(pl.program_id(0) == 0)
    def _():
        mx_ref[...] = m

    @pl.when(pl.program_id(0) != 0)
    def _():
        mx_ref[...] = jnp.maximum(mx_ref[...], m)


def _tc2(p0, p1, b1r, W2, A2):
    return pl.pallas_call(
        _tc2_body,
        grid=(GRID,),
        in_specs=[
            pl.BlockSpec((BM, HEADS * HID), lambda i: (i, 0)),
            pl.BlockSpec((BM, HEADS * HID), lambda i: (i, 0)),
            pl.BlockSpec((1, HEADS * HID), lambda i: (0, 0)),
            pl.BlockSpec((HEADS * HID, OUT), lambda i: (0, 0)),
            pl.BlockSpec((OUT, 16), lambda i: (0, 0)),
        ],
        out_specs=[
            pl.BlockSpec((BM, OUT), lambda i: (i, 0)),
            pl.BlockSpec((BM, 16), lambda i: (i, 0)),
            pl.BlockSpec((1, 16), lambda i: (0, 0)),
        ],
        out_shape=[
            jax.ShapeDtypeStruct((NP, OUT), jnp.float32),
            jax.ShapeDtypeStruct((NP, 16), jnp.float32),
            jax.ShapeDtypeStruct((1, 16), jnp.float32),
        ],
    )(p0, p1, b1r, W2, A2)


def _tc3_body(q0_ref, q1_ref, b_ref, o_ref):
    o = q0_ref[...] + q1_ref[...] + b_ref[...]
    m = jnp.max(o, axis=1, keepdims=True)
    z = o - m
    lse = jnp.log(jnp.sum(jnp.exp(z), axis=1, keepdims=True))
    o_ref[...] = z - lse


def _tc3(q0, q1, b2r):
    return pl.pallas_call(
        _tc3_body,
        grid=(GRID,),
        in_specs=[
            pl.BlockSpec((BM, OUT), lambda i: (i, 0)),
            pl.BlockSpec((BM, OUT), lambda i: (i, 0)),
            pl.BlockSpec((1, OUT), lambda i: (0, 0)),
        ],
        out_specs=pl.BlockSpec((BM, OUT), lambda i: (i, 0)),
        out_shape=jax.ShapeDtypeStruct((NP, OUT), jnp.float32),
    )(q0, q1, b2r)


def _tcradd_body(a_ref, b_ref, o_ref):
    o_ref[...] = 1.0 / (a_ref[...] + b_ref[...] + 1e-16)


def _tc_radd(a, b):
    rows, cols = a.shape
    bm = min(rows, BM)
    return pl.pallas_call(
        _tcradd_body,
        grid=(rows // bm,),
        in_specs=[
            pl.BlockSpec((bm, cols), lambda i: (i, 0)),
            pl.BlockSpec((bm, cols), lambda i: (i, 0)),
        ],
        out_specs=pl.BlockSpec((bm, cols), lambda i: (i, 0)),
        out_shape=jax.ShapeDtypeStruct((rows, cols), jnp.float32),
    )(a, b)


@functools.partial(
    pl.kernel,
    out_type=[
        jax.ShapeDtypeStruct((NC, NP, 16), jnp.float32),
        jax.ShapeDtypeStruct((NCHUNK_P, B, 16), jnp.float32),
    ],
    mesh=_mesh,
    compiler_params=_scparams,
    scratch_types=[
        pltpu.VMEM((CPT, B), jnp.int32),
        pltpu.VMEM((CPT, B), jnp.int32),
        pltpu.VMEM((B, 16), jnp.float32),
        pltpu.VMEM((B, 16), jnp.float32),
        pltpu.VMEM((B, 16), jnp.float32),
        pltpu.VMEM((B, 16), jnp.float32),
        pltpu.VMEM((B, 16), jnp.float32),
        pltpu.VMEM((B, 16), jnp.float32),
        pltpu.VMEM((L,), jnp.float32),
        pltpu.VMEM_SHARED((NP, 16), jnp.float32),
        pltpu.SemaphoreType.DMA,
        pltpu.SemaphoreType.DMA,
        pltpu.SemaphoreType.DMA,
        pltpu.SemaphoreType.DMA,
        pltpu.SemaphoreType.DMA,
        pltpu.SemaphoreType.DMA,
    ],
)
def _sc1_pass1(src_hbm, dst_hbm, ta_hbm, tb_hbm, k_hbm, z_hbm, den_hbm,
               ex_hbm,
               src_v, dst_v, sa0, sa1, sb0, sb1, ex0, ex1, k_v, den_sp,
               sma0, sma1, smb0, smb1, sme0, sme1):
    c = lax.axis_index("c")
    s = lax.axis_index("s")
    wid = c * NS + s
    base = wid * CPT
    pltpu.sync_copy(k_hbm, k_v)
    kv = k_v[...]
    pltpu.sync_copy(z_hbm.at[pl.ds(s * ROWS, ROWS), :],
                    den_sp.at[pl.ds(s * ROWS, ROWS), :])
    pltpu.sync_copy(src_hbm.at[pl.ds(wid * CPT, CPT), :], src_v)
    pltpu.sync_copy(dst_hbm.at[pl.ds(wid * CPT, CPT), :], dst_v)
    plsc.subcore_barrier()

    bufs = ((sa0, sb0, ex0, sma0, smb0, sme0),
            (sa1, sb1, ex1, sma1, smb1, sme1))
    for par in range(2):
        sa, sb, ex, sma, smb, sme = bufs[par]
        pltpu.async_copy(ta_hbm.at[src_v.at[par]], sa, sma)
        pltpu.async_copy(tb_hbm.at[dst_v.at[par]], sb, smb)

    def pair(jj, carry):
        for par in range(2):
            j = jj * 2 + par
            sa, sb, ex, sma, smb, sme = bufs[par]
            pltpu.make_async_copy(ta_hbm.at[src_v.at[j]], sa, sma).wait()
            pltpu.make_async_copy(tb_hbm.at[dst_v.at[j]], sb, smb).wait()

            @pl.when(j >= 2)
            def _():
                pltpu.make_async_copy(ex, ex_hbm.at[j - 2 + base], sme).wait()

            @plsc.parallel_loop(0, B, 1, unroll=4)
            def _(i):
                e = sa[i, :] + sb[i, :]
                e = jnp.maximum(e, 0.2 * e)
                ex[i, :] = jnp.exp(e - kv)

            pltpu.sync_copy(ex, den_sp.at[dst_v.at[j]], add=True)
            pltpu.async_copy(ex, ex_hbm.at[j + base], sme)

            @pl.when(j + 2 < CPT)
            def _():
                pltpu.async_copy(ta_hbm.at[src_v.at[j + 2]], sa, sma)
                pltpu.async_copy(tb_hbm.at[dst_v.at[j + 2]], sb, smb)

        return carry

    lax.fori_loop(0, CPT // 2, pair, 0)
    for par in range(2):
        sa, sb, ex, sma, smb, sme = bufs[par]
        pltpu.make_async_copy(
            ex, ex_hbm.at[CPT - 2 + par + base], sme).wait()
    plsc.subcore_barrier()
    pltpu.sync_copy(den_sp.at[pl.ds(s * ROWS, ROWS), :],
                    den_hbm.at[c, pl.ds(s * ROWS, ROWS), :])


@functools.partial(
    pl.kernel,
    out_type=jax.ShapeDtypeStruct((NC, NP, HEADS * HID), jnp.float32),
    mesh=_mesh,
    compiler_params=_scparams,
    scratch_types=[
        pltpu.VMEM((1, B), jnp.int32),
        pltpu.VMEM((1, B), jnp.int32),
        pltpu.VMEM((1, B), jnp.int32),
        pltpu.VMEM((1, B), jnp.int32),
        pltpu.VMEM((B, 16), jnp.float32),
        pltpu.VMEM((B, 16), jnp.float32),
        pltpu.VMEM((B, 16), jnp.float32),
        pltpu.VMEM((B, 16), jnp.float32),
        pltpu.VMEM((B, HEADS * HID), jnp.float32),
        pltpu.VMEM((B, HEADS * HID), jnp.float32),
        pltpu.VMEM_SHARED((NP, HEADS * HID), jnp.float32),
        pltpu.SemaphoreType.DMA,
        pltpu.SemaphoreType.DMA,
        pltpu.SemaphoreType.DMA,
        pltpu.SemaphoreType.DMA,
        pltpu.SemaphoreType.DMA,
        pltpu.SemaphoreType.DMA,
        pltpu.SemaphoreType.DMA,
        pltpu.SemaphoreType.DMA,
        pltpu.SemaphoreType.DMA,
        pltpu.SemaphoreType.DMA,
    ],
)
def _sc1_pass2(src_hbm, dst_hbm, ex_hbm, den_hbm, h_hbm, z_hbm, out_hbm,
               s0, s1, d0, d1, ex0, ex1, dn0, dn1, h0, h1,
               out_sp,
               si0, si1, di0, di1, sme0, sme1, smd0, smd1, smh0, smh1):
    c = lax.axis_index("c")
    s = lax.axis_index("s")
    wid = c * NS + s
    base = wid * CPT
    mod8 = lax.iota(jnp.int32, L) % HEADS
    pltpu.sync_copy(z_hbm.at[pl.ds(s * ROWS, ROWS), :],
                    out_sp.at[pl.ds(s * ROWS, ROWS), :])
    plsc.subcore_barrier()

    ibufs = ((s0, d0, si0, di0), (s1, d1, si1, di1))
    rbufs = ((ex0, dn0, h0, sme0, smd0, smh0),
             (ex1, dn1, h1, sme1, smd1, smh1))

    def idx_copy(j, par):
        sv, dv, sis, dis = ibufs[par]
        pltpu.async_copy(src_hbm.at[pl.ds(base + j, 1), :], sv, sis)
        pltpu.async_copy(dst_hbm.at[pl.ds(base + j, 1), :], dv, dis)

    def idx_wait(j, par):
        sv, dv, sis, dis = ibufs[par]
        pltpu.make_async_copy(
            src_hbm.at[pl.ds(base + j, 1), :], sv, sis).wait()
        pltpu.make_async_copy(
            dst_hbm.at[pl.ds(base + j, 1), :], dv, dis).wait()

    def ex_copy(j, par):
        ex, _, _, sme, _, _ = rbufs[par]
        pltpu.async_copy(ex_hbm.at[base + j], ex, sme)

    def row_issue(par):
        sv, dv, _, _ = ibufs[par]
        ex, dn, h, sme, smd, smh = rbufs[par]
        pltpu.async_copy(den_hbm.at[dv.at[0]], dn, smd)
        pltpu.async_copy(h_hbm.at[sv.at[0]], h, smh)

    def row_wait(j, par):
        sv, dv, _, _ = ibufs[par]
        ex, dn, h, sme, smd, smh = rbufs[par]
        pltpu.make_async_copy(ex_hbm.at[base + j], ex, sme).wait()
        pltpu.make_async_copy(den_hbm.at[dv.at[0]], dn, smd).wait()
        pltpu.make_async_copy(h_hbm.at[sv.at[0]], h, smh).wait()

    idx_copy(0, 0)
    ex_copy(0, 0)
    ex_copy(1, 1)
    idx_wait(0, 0)
    row_issue(0)
    idx_copy(1, 1)

    def pair(jj, carry):
        for par in range(2):
            j = jj * 2 + par

            @pl.when(j + 1 < CPT)
            def _():
                idx_wait(j + 1, par ^ 1)
                row_issue(par ^ 1)

            row_wait(j, par)
            ex, dn, h = rbufs[par][:3]

            @plsc.parallel_loop(0, B, 1, unroll=2)
            def _(i):
                al = ex[i, :] * dn[i, :]
                av = _bcast(al, mod8)
                for k in range(HEADS):
                    h[i, pl.ds(k * HID, HID)] = (
                        h[i, pl.ds(k * HID, HID)] * av)

            dv = ibufs[par][1]
            pltpu.sync_copy(h, out_sp.at[dv.at[0]], add=True)

            @pl.when(j + 2 < CPT)
            def _():
                idx_copy(j + 2, par)
                ex_copy(j + 2, par)

        return carry

    lax.fori_loop(0, CPT // 2, pair, 0)
    plsc.subcore_barrier()
    pltpu.sync_copy(out_sp.at[pl.ds(s * ROWS, ROWS), :],
                    out_hbm.at[c, pl.ds(s * ROWS, ROWS), :])


@functools.partial(
    pl.kernel,
    out_type=[
        jax.ShapeDtypeStruct((NC, NP), jnp.float32),
        jax.ShapeDtypeStruct((NCHUNK_P, B), jnp.float32),
    ],
    mesh=_mesh,
    compiler_params=_scparams,
    scratch_types=[
        pltpu.VMEM((CPT, B), jnp.int32),
        pltpu.VMEM((CPT, B), jnp.int32),
        pltpu.VMEM((CPT, B), jnp.float32),
        pltpu.VMEM((NP,), jnp.float32),
        pltpu.VMEM((NP,), jnp.float32),
        pltpu.VMEM((L,), jnp.float32),
        pltpu.VMEM_SHARED((NP,), jnp.float32),
    ],
)
def _sc2_pass1(src_hbm, dst_hbm, as_hbm, ad_hbm, k_hbm, z_hbm, den_hbm,
               ex_hbm,
               src_v, dst_v, exs_v, as_v, ad_v, k_v, den_sp):
    c = lax.axis_index("c")
    s = lax.axis_index("s")
    wid = c * NS + s
    pltpu.sync_copy(k_hbm, k_v)
    kv = k_v[...]
    pltpu.sync_copy(as_hbm, as_v)
    pltpu.sync_copy(ad_hbm, ad_v)
    pltpu.sync_copy(z_hbm.at[pl.ds(s * ROWS, ROWS)],
                    den_sp.at[pl.ds(s * ROWS, ROWS)])
    pltpu.sync_copy(src_hbm.at[pl.ds(wid * CPT, CPT), :], src_v)
    pltpu.sync_copy(dst_hbm.at[pl.ds(wid * CPT, CPT), :], dst_v)
    plsc.subcore_barrier()

    def chunk(j, carry):

        @plsc.parallel_loop(0, B // L, 1, unroll=4)
        def _(j2):
            sl = pl.ds(j2 * L, L)
            sv = plsc.load_gather(as_v, [src_v[j, sl]])
            dv = plsc.load_gather(ad_v, [dst_v[j, sl]])
            e = sv + dv
            e = jnp.maximum(e, 0.2 * e)
            exs_v[j, sl] = jnp.exp(e - kv)

        pltpu.sync_copy(exs_v.at[j], den_sp.at[dst_v.at[j]], add=True)
        return carry

    lax.fori_loop(0, CPT, chunk, 0)
    pltpu.sync_copy(exs_v, ex_hbm.at[pl.ds(wid * CPT, CPT), :])
    plsc.subcore_barrier()
    pltpu.sync_copy(den_sp.at[pl.ds(s * ROWS, ROWS)],
                    den_hbm.at[c, pl.ds(s * ROWS, ROWS)])


@functools.partial(
    pl.kernel,
    out_type=jax.ShapeDtypeStruct((NC, NP, OUT), jnp.float32),
    mesh=_mesh,
    compiler_params=_scparams,
    scratch_types=[
        pltpu.VMEM((CPT, B), jnp.int32),
        pltpu.VMEM((CPT, B), jnp.int32),
        pltpu.VMEM((CPT, B), jnp.float32),
        pltpu.VMEM((B,), jnp.float32),
        pltpu.VMEM((NP,), jnp.float32),
        pltpu.VMEM((B, OUT), jnp.float32),
        pltpu.VMEM((B, OUT), jnp.float32),
        pltpu.VMEM((B, OUT), jnp.float32),
        pltpu.VMEM_SHARED((NP, OUT), jnp.float32),
        pltpu.SemaphoreType.DMA,
        pltpu.SemaphoreType.DMA,
    ],
)
def _sc2_pass2(src_hbm, dst_hbm, ex_hbm, den_hbm, h_hbm, z_hbm, out_hbm,
               src_v, dst_v, exs_v, al_v, dn_v, h0, h1, msg_v,
               out_sp, smh0, smh1):
    c = lax.axis_index("c")
    s = lax.axis_index("s")
    wid = c * NS + s
    pltpu.sync_copy(den_hbm, dn_v)
    pltpu.sync_copy(z_hbm.at[pl.ds(s * ROWS, ROWS), :],
                    out_sp.at[pl.ds(s * ROWS, ROWS), :])
    pltpu.sync_copy(src_hbm.at[pl.ds(wid * CPT, CPT), :], src_v)
    pltpu.sync_copy(dst_hbm.at[pl.ds(wid * CPT, CPT), :], dst_v)
    pltpu.sync_copy(ex_hbm.at[pl.ds(wid * CPT, CPT), :], exs_v)
    plsc.subcore_barrier()

    bufs = ((h0, smh0), (h1, smh1))
    for par in range(2):
        h, smh = bufs[par]
        pltpu.async_copy(h_hbm.at[src_v.at[par]], h, smh)

    def pair(jj, carry):
        for par in range(2):
            j = jj * 2 + par
            h, smh = bufs[par]

            @plsc.parallel_loop(0, B // L, 1, unroll=4)
            def _(j2):
                sl = pl.ds(j2 * L, L)
                den = plsc.load_gather(dn_v, [dst_v[j, sl]])
                al_v[sl] = exs_v[j, sl] * den

            pltpu.make_async_copy(h_hbm.at[src_v.at[j]], h, smh).wait()

            @plsc.parallel_loop(0, B // L, 1, unroll=1)
            def _(g):
                avall = al_v[pl.ds(g * L, L)]
                for m in range(L):
                    av = _bcast(avall, jnp.full((L,), m, jnp.int32))
                    i = g * L + m
                    for k in range(OUT // HID):
                        msg_v[i, pl.ds(k * HID, HID)] = (
                            h[i, pl.ds(k * HID, HID)] * av)

            pltpu.sync_copy(msg_v, out_sp.at[dst_v.at[j]], add=True)

            @pl.when(j + 2 < CPT)
            def _():
                pltpu.async_copy(h_hbm.at[src_v.at[j + 2]], h, smh)

        return carry

    lax.fori_loop(0, CPT // 2, pair, 0)
    plsc.subcore_barrier()
    pltpu.sync_copy(out_sp.at[pl.ds(s * ROWS, ROWS), :],
                    out_hbm.at[c, pl.ds(s * ROWS, ROWS), :])


def kernel(x, edge_index, W1, a_src1, a_dst1, b1, W2, a_src2, a_dst2, b2):
    x_pad = jnp.pad(x, ((0, NP - N), (0, 0)))
    src = edge_index[0].astype(jnp.int32)
    dst = edge_index[1].astype(jnp.int32)
    pad = N + jnp.arange(EP - E, dtype=jnp.int32) % (NP - N)
    src_p = jnp.concatenate([src, pad]).reshape(NCHUNK_P, B)
    dst_p = jnp.concatenate([dst, pad]).reshape(NCHUNK_P, B)

    cols = jnp.arange(HEADS * HID)
    orig = (cols % HEADS) * HID + cols // HEADS
    W1p = W1[:, orig]
    b1p = b1[orig]
    W2p = W2[orig, :]

    eye = jnp.repeat(jnp.eye(HEADS, dtype=jnp.float32), HID, axis=0)
    A_src = eye * a_src1.reshape(-1)[:, None]
    A_dst = eye * a_dst1.reshape(-1)[:, None]
    A1 = jnp.concatenate([A_src, A_dst], axis=1)[orig, :]

    h1, ta1, tb1, mx1 = _tc1(x_pad, W1p, A1)
    k1 = mx1[0, :8] + mx1[0, 8:]
    k1 = jnp.maximum(k1, 0.2 * k1)
    k16_1 = jnp.concatenate([k1, k1])

    z16 = jnp.zeros((NP, 16), jnp.float32)
    z128 = jnp.zeros((NP, HEADS * HID), jnp.float32)
    den1p, ex1 = _sc1_pass1(src_p, dst_p, ta1, tb1, k16_1, z16)
    rden1 = _tc_radd(den1p[0], den1p[1])
    out1p = _sc1_pass2(src_p, dst_p, ex1, rden1, h1, z128)

    A2 = jnp.zeros((OUT, 16), jnp.float32)
    A2 = A2.at[:, 0].set(a_src2[0]).at[:, 1].set(a_dst2[0])
    h2, sa2, mx2 = _tc2(out1p[0], out1p[1], b1p.reshape(1, -1), W2p, A2)
    k2 = mx2[0, 0] + mx2[0, 1]
    k2 = jnp.maximum(k2, 0.2 * k2)
    k16_2 = jnp.full((L,), k2, jnp.float32)
    as2 = sa2[:, 0]
    ad2 = sa2[:, 1]

    z1d = jnp.zeros((NP,), jnp.float32)
    z64 = jnp.zeros((NP, OUT), jnp.float32)
    den2p, ex2 = _sc2_pass1(src_p, dst_p, as2, ad2, k16_2, z1d)
    rden2 = _tc_radd(den2p[0].reshape(80, 128),
                     den2p[1].reshape(80, 128)).reshape(NP)
    out2p = _sc2_pass2(src_p, dst_p, ex2, rden2, h2, z64)

    o = _tc3(out2p[0], out2p[1], b2.reshape(1, -1))
    return o[:N]


# speedup vs baseline: 1.8915x; 1.0271x over previous
import functools

import jax
import jax.numpy as jnp
from jax import lax
from jax.experimental import pallas as pl
from jax.experimental.pallas import tpu as pltpu
from jax.experimental.pallas import tpu_sc as plsc

N = 10000
E = 320000
IN = 128
HID = 16
HEADS = 8
OUT = 64

NP = 10240
BM = 256
GRID = NP // BM

NC = 2
NS = 16
L = 16
B = 128
CPT = 80
NCHUNK_P = NC * NS * CPT
EP = NCHUNK_P * B
B2 = 256
CPT2 = 40
NCHUNK_P2 = NC * NS * CPT2
ROWS = NP // NS

_mesh = plsc.VectorSubcoreMesh(
    core_axis_name="c", subcore_axis_name="s", num_cores=NC, num_subcores=NS)
_scparams = pltpu.CompilerParams(
    use_tc_tiling_on_sc=False, needs_layout_passes=False)


def _bcast(v, idx):
    dn = lax.GatherDimensionNumbers(
        offset_dims=(), collapsed_slice_dims=(0,), start_index_map=(0,))
    return lax.gather(v, idx[:, None], dn, (1,),
                      mode=lax.GatherScatterMode.PROMISE_IN_BOUNDS)


def _tc1_body(x_ref, w_ref, a_ref, h_ref, sa_ref, sb_ref, mx_ref):
    h = jnp.dot(x_ref[...], w_ref[...], preferred_element_type=jnp.float32)
    h_ref[...] = h
    s = jnp.dot(h, a_ref[...], preferred_element_type=jnp.float32)
    sa_ref[...] = s
    sb_ref[...] = jnp.concatenate([s[:, 8:], s[:, :8]], axis=1)
    m = jnp.max(s, axis=0, keepdims=True)

    @pl.when(pl.program_id(0) == 0)
    def _():
        mx_ref[...] = m

    @pl.when(pl.program_id(0) != 0)
    def _():
        mx_ref[...] = jnp.maximum(mx_ref[...], m)


def _tc1(x_pad, W1, A1):
    return pl.pallas_call(
        _tc1_body,
        grid=(GRID,),
        in_specs=[
            pl.BlockSpec((BM, IN), lambda i: (i, 0)),
            pl.BlockSpec((IN, HEADS * HID), lambda i: (0, 0)),
            pl.BlockSpec((HEADS * HID, 16), lambda i: (0, 0)),
        ],
        out_specs=[
            pl.BlockSpec((BM, HEADS * HID), lambda i: (i, 0)),
            pl.BlockSpec((BM, 16), lambda i: (i, 0)),
            pl.BlockSpec((BM, 16), lambda i: (i, 0)),
            pl.BlockSpec((1, 16), lambda i: (0, 0)),
        ],
        out_shape=[
            jax.ShapeDtypeStruct((NP, HEADS * HID), jnp.float32),
            jax.ShapeDtypeStruct((NP, 16), jnp.float32),
            jax.ShapeDtypeStruct((NP, 16), jnp.float32),
            jax.ShapeDtypeStruct((1, 16), jnp.float32),
        ],
    )(x_pad, W1, A1)


def _tc2_body(p0_ref, p1_ref, b_ref, w_ref, a_ref, h_ref, sa_ref, mx_ref):
    t = jnp.maximum(p0_ref[...] + p1_ref[...] + b_ref[...], 0.0)
    h = jnp.dot(t, w_ref[...], preferred_element_type=jnp.float32)
    h_ref[...] = h
    s = jnp.dot(h, a_ref[...], preferred_element_type=jnp.float32)
    sa_ref[...] = s
    m = jnp.max(s, axis=0, keepdims=True)

    @pl.when(pl.program_id(0) == 0)
    def _():
        mx_ref[...] = m

    @pl.when(pl.program_id(0) != 0)
    def _():
        mx_ref[...] = jnp.maximum(mx_ref[...], m)


def _tc2(p0, p1, b1r, W2, A2):
    return pl.pallas_call(
        _tc2_body,
        grid=(GRID,),
        in_specs=[
            pl.BlockSpec((BM, HEADS * HID), lambda i: (i, 0)),
            pl.BlockSpec((BM, HEADS * HID), lambda i: (i, 0)),
            pl.BlockSpec((1, HEADS * HID), lambda i: (0, 0)),
            pl.BlockSpec((HEADS * HID, OUT), lambda i: (0, 0)),
            pl.BlockSpec((OUT, 16), lambda i: (0, 0)),
        ],
        out_specs=[
            pl.BlockSpec((BM, OUT), lambda i: (i, 0)),
            pl.BlockSpec((BM, 16), lambda i: (i, 0)),
            pl.BlockSpec((1, 16), lambda i: (0, 0)),
        ],
        out_shape=[
            jax.ShapeDtypeStruct((NP, OUT), jnp.float32),
            jax.ShapeDtypeStruct((NP, 16), jnp.float32),
            jax.ShapeDtypeStruct((1, 16), jnp.float32),
        ],
    )(p0, p1, b1r, W2, A2)


def _tc3_body(q0_ref, q1_ref, b_ref, o_ref):
    o = q0_ref[...] + q1_ref[...] + b_ref[...]
    m = jnp.max(o, axis=1, keepdims=True)
    z = o - m
    lse = jnp.log(jnp.sum(jnp.exp(z), axis=1, keepdims=True))
    o_ref[...] = z - lse


def _tc3(q0, q1, b2r):
    return pl.pallas_call(
        _tc3_body,
        grid=(GRID,),
        in_specs=[
            pl.BlockSpec((BM, OUT), lambda i: (i, 0)),
            pl.BlockSpec((BM, OUT), lambda i: (i, 0)),
            pl.BlockSpec((1, OUT), lambda i: (0, 0)),
        ],
        out_specs=pl.BlockSpec((BM, OUT), lambda i: (i, 0)),
        out_shape=jax.ShapeDtypeStruct((NP, OUT), jnp.float32),
    )(q0, q1, b2r)


def _tcradd_body(a_ref, b_ref, o_ref):
    o_ref[...] = 1.0 / (a_ref[...] + b_ref[...] + 1e-16)


def _tc_radd(a, b):
    rows, cols = a.shape
    bm = min(rows, BM)
    return pl.pallas_call(
        _tcradd_body,
        grid=(rows // bm,),
        in_specs=[
            pl.BlockSpec((bm, cols), lambda i: (i, 0)),
            pl.BlockSpec((bm, cols), lambda i: (i, 0)),
        ],
        out_specs=pl.BlockSpec((bm, cols), lambda i: (i, 0)),
        out_shape=jax.ShapeDtypeStruct((rows, cols), jnp.float32),
    )(a, b)


@functools.partial(
    pl.kernel,
    out_type=[
        jax.ShapeDtypeStruct((NC, NP, 16), jnp.float32),
        jax.ShapeDtypeStruct((EP, 16), jnp.float32),
    ],
    mesh=_mesh,
    compiler_params=_scparams,
    scratch_types=[
        pltpu.VMEM((CPT2, B2), jnp.int32),
        pltpu.VMEM((CPT2, B2), jnp.int32),
        pltpu.VMEM((B2, 16), jnp.float32),
        pltpu.VMEM((B2, 16), jnp.float32),
        pltpu.VMEM((B2, 16), jnp.float32),
        pltpu.VMEM((B2, 16), jnp.float32),
        pltpu.VMEM((B2, 16), jnp.float32),
        pltpu.VMEM((B2, 16), jnp.float32),
        pltpu.VMEM((L,), jnp.float32),
        pltpu.VMEM_SHARED((NP, 16), jnp.float32),
        pltpu.SemaphoreType.DMA,
        pltpu.SemaphoreType.DMA,
        pltpu.SemaphoreType.DMA,
        pltpu.SemaphoreType.DMA,
        pltpu.SemaphoreType.DMA,
        pltpu.SemaphoreType.DMA,
    ],
)
def _sc1_pass1(src_hbm, dst_hbm, ta_hbm, tb_hbm, k_hbm, z_hbm, den_hbm,
               ex_hbm,
               src_v, dst_v, sa0, sa1, sb0, sb1, ex0, ex1, k_v, den_sp,
               sma0, sma1, smb0, smb1, sme0, sme1):
    c = lax.axis_index("c")
    s = lax.axis_index("s")
    wid = c * NS + s
    base = wid * CPT2
    pltpu.sync_copy(k_hbm, k_v)
    kv = k_v[...]
    pltpu.sync_copy(z_hbm.at[pl.ds(s * ROWS, ROWS), :],
                    den_sp.at[pl.ds(s * ROWS, ROWS), :])
    pltpu.sync_copy(src_hbm.at[pl.ds(wid * CPT2, CPT2), :], src_v)
    pltpu.sync_copy(dst_hbm.at[pl.ds(wid * CPT2, CPT2), :], dst_v)
    plsc.subcore_barrier()

    bufs = ((sa0, sb0, ex0, sma0, smb0, sme0),
            (sa1, sb1, ex1, sma1, smb1, sme1))
    for par in range(2):
        sa, sb, ex, sma, smb, sme = bufs[par]
        pltpu.async_copy(ta_hbm.at[src_v.at[par]], sa, sma)
        pltpu.async_copy(tb_hbm.at[dst_v.at[par]], sb, smb)

    def pair(jj, carry):
        for par in range(2):
            j = jj * 2 + par
            sa, sb, ex, sma, smb, sme = bufs[par]
            pltpu.make_async_copy(ta_hbm.at[src_v.at[j]], sa, sma).wait()
            pltpu.make_async_copy(tb_hbm.at[dst_v.at[j]], sb, smb).wait()

            @pl.when(j >= 2)
            def _():
                pltpu.make_async_copy(
                    ex, ex_hbm.at[pl.ds((base + j - 2) * B2, B2), :],
                    sme).wait()

            @plsc.parallel_loop(0, B2, 1, unroll=4)
            def _(i):
                e = sa[i, :] + sb[i, :]
                e = jnp.maximum(e, 0.2 * e)
                ex[i, :] = jnp.exp(e - kv)

            pltpu.sync_copy(ex, den_sp.at[dst_v.at[j]], add=True)
            pltpu.async_copy(
                ex, ex_hbm.at[pl.ds((base + j) * B2, B2), :], sme)

            @pl.when(j + 2 < CPT2)
            def _():
                pltpu.async_copy(ta_hbm.at[src_v.at[j + 2]], sa, sma)
                pltpu.async_copy(tb_hbm.at[dst_v.at[j + 2]], sb, smb)

        return carry

    lax.fori_loop(0, CPT2 // 2, pair, 0)
    for par in range(2):
        sa, sb, ex, sma, smb, sme = bufs[par]
        pltpu.make_async_copy(
            ex, ex_hbm.at[pl.ds((base + CPT2 - 2 + par) * B2, B2), :],
            sme).wait()
    plsc.subcore_barrier()
    pltpu.sync_copy(den_sp.at[pl.ds(s * ROWS, ROWS), :],
                    den_hbm.at[c, pl.ds(s * ROWS, ROWS), :])


@functools.partial(
    pl.kernel,
    out_type=jax.ShapeDtypeStruct((NC, NP, HEADS * HID), jnp.float32),
    mesh=_mesh,
    compiler_params=_scparams,
    scratch_types=[
        pltpu.VMEM((1, B), jnp.int32),
        pltpu.VMEM((1, B), jnp.int32),
        pltpu.VMEM((1, B), jnp.int32),
        pltpu.VMEM((1, B), jnp.int32),
        pltpu.VMEM((B, 16), jnp.float32),
        pltpu.VMEM((B, 16), jnp.float32),
        pltpu.VMEM((B, 16), jnp.float32),
        pltpu.VMEM((B, 16), jnp.float32),
        pltpu.VMEM((B, HEADS * HID), jnp.float32),
        pltpu.VMEM((B, HEADS * HID), jnp.float32),
        pltpu.VMEM_SHARED((NP, HEADS * HID), jnp.float32),
        pltpu.SemaphoreType.DMA,
        pltpu.SemaphoreType.DMA,
        pltpu.SemaphoreType.DMA,
        pltpu.SemaphoreType.DMA,
        pltpu.SemaphoreType.DMA,
        pltpu.SemaphoreType.DMA,
        pltpu.SemaphoreType.DMA,
        pltpu.SemaphoreType.DMA,
        pltpu.SemaphoreType.DMA,
        pltpu.SemaphoreType.DMA,
    ],
)
def _sc1_pass2(src_hbm, dst_hbm, ex_hbm, den_hbm, h_hbm, z_hbm, out_hbm,
               s0, s1, d0, d1, ex0, ex1, dn0, dn1, h0, h1,
               out_sp,
               si0, si1, di0, di1, sme0, sme1, smd0, smd1, smh0, smh1):
    c = lax.axis_index("c")
    s = lax.axis_index("s")
    wid = c * NS + s
    base = wid * CPT
    mod8 = lax.iota(jnp.int32, L) % HEADS
    pltpu.sync_copy(z_hbm.at[pl.ds(s * ROWS, ROWS), :],
                    out_sp.at[pl.ds(s * ROWS, ROWS), :])
    plsc.subcore_barrier()

    ibufs = ((s0, d0, si0, di0), (s1, d1, si1, di1))
    rbufs = ((ex0, dn0, h0, sme0, smd0, smh0),
             (ex1, dn1, h1, sme1, smd1, smh1))

    def idx_copy(j, par):
        sv, dv, sis, dis = ibufs[par]
        pltpu.async_copy(src_hbm.at[pl.ds(base + j, 1), :], sv, sis)
        pltpu.async_copy(dst_hbm.at[pl.ds(base + j, 1), :], dv, dis)

    def idx_wait(j, par):
        sv, dv, sis, dis = ibufs[par]
        pltpu.make_async_copy(
            src_hbm.at[pl.ds(base + j, 1), :], sv, sis).wait()
        pltpu.make_async_copy(
            dst_hbm.at[pl.ds(base + j, 1), :], dv, dis).wait()

    def ex_copy(j, par):
        ex, _, _, sme, _, _ = rbufs[par]
        pltpu.async_copy(ex_hbm.at[pl.ds((base + j) * B, B), :], ex, sme)

    def row_issue(par):
        sv, dv, _, _ = ibufs[par]
        ex, dn, h, sme, smd, smh = rbufs[par]
        pltpu.async_copy(den_hbm.at[dv.at[0]], dn, smd)
        pltpu.async_copy(h_hbm.at[sv.at[0]], h, smh)

    def row_wait(j, par):
        sv, dv, _, _ = ibufs[par]
        ex, dn, h, sme, smd, smh = rbufs[par]
        pltpu.make_async_copy(
            ex_hbm.at[pl.ds((base + j) * B, B), :], ex, sme).wait()
        pltpu.make_async_copy(den_hbm.at[dv.at[0]], dn, smd).wait()
        pltpu.make_async_copy(h_hbm.at[sv.at[0]], h, smh).wait()

    idx_copy(0, 0)
    ex_copy(0, 0)
    ex_copy(1, 1)
    idx_wait(0, 0)
    row_issue(0)
    idx_copy(1, 1)

    def pair(jj, carry):
        for par in range(2):
            j = jj * 2 + par

            @pl.when(j + 1 < CPT)
            def _():
                idx_wait(j + 1, par ^ 1)
                row_issue(par ^ 1)

            row_wait(j, par)
            ex, dn, h = rbufs[par][:3]

            @plsc.parallel_loop(0, B, 1, unroll=2)
            def _(i):
                al = ex[i, :] * dn[i, :]
                av = _bcast(al, mod8)
                for k in range(HEADS):
                    h[i, pl.ds(k * HID, HID)] = (
                        h[i, pl.ds(k * HID, HID)] * av)

            dv = ibufs[par][1]
            pltpu.sync_copy(h, out_sp.at[dv.at[0]], add=True)

            @pl.when(j + 2 < CPT)
            def _():
                idx_copy(j + 2, par)
                ex_copy(j + 2, par)

        return carry

    lax.fori_loop(0, CPT // 2, pair, 0)
    plsc.subcore_barrier()
    pltpu.sync_copy(out_sp.at[pl.ds(s * ROWS, ROWS), :],
                    out_hbm.at[c, pl.ds(s * ROWS, ROWS), :])


@functools.partial(
    pl.kernel,
    out_type=[
        jax.ShapeDtypeStruct((NC, NP), jnp.float32),
        jax.ShapeDtypeStruct((NCHUNK_P, B), jnp.float32),
    ],
    mesh=_mesh,
    compiler_params=_scparams,
    scratch_types=[
        pltpu.VMEM((CPT, B), jnp.int32),
        pltpu.VMEM((CPT, B), jnp.int32),
        pltpu.VMEM((CPT, B), jnp.float32),
        pltpu.VMEM((NP,), jnp.float32),
        pltpu.VMEM((NP,), jnp.float32),
        pltpu.VMEM((L,), jnp.float32),
        pltpu.VMEM_SHARED((NP,), jnp.float32),
    ],
)
def _sc2_pass1(src_hbm, dst_hbm, as_hbm, ad_hbm, k_hbm, z_hbm, den_hbm,
               ex_hbm,
               src_v, dst_v, exs_v, as_v, ad_v, k_v, den_sp):
    c = lax.axis_index("c")
    s = lax.axis_index("s")
    wid = c * NS + s
    pltpu.sync_copy(k_hbm, k_v)
    kv = k_v[...]
    pltpu.sync_copy(as_hbm, as_v)
    pltpu.sync_copy(ad_hbm, ad_v)
    pltpu.sync_copy(z_hbm.at[pl.ds(s * ROWS, ROWS)],
                    den_sp.at[pl.ds(s * ROWS, ROWS)])
    pltpu.sync_copy(src_hbm.at[pl.ds(wid * CPT, CPT), :], src_v)
    pltpu.sync_copy(dst_hbm.at[pl.ds(wid * CPT, CPT), :], dst_v)
    plsc.subcore_barrier()

    def chunk(j, carry):

        @plsc.parallel_loop(0, B // L, 1, unroll=4)
        def _(j2):
            sl = pl.ds(j2 * L, L)
            sv = plsc.load_gather(as_v, [src_v[j, sl]])
            dv = plsc.load_gather(ad_v, [dst_v[j, sl]])
            e = sv + dv
            e = jnp.maximum(e, 0.2 * e)
            exs_v[j, sl] = jnp.exp(e - kv)

        pltpu.sync_copy(exs_v.at[j], den_sp.at[dst_v.at[j]], add=True)
        return carry

    lax.fori_loop(0, CPT, chunk, 0)
    pltpu.sync_copy(exs_v, ex_hbm.at[pl.ds(wid * CPT, CPT), :])
    plsc.subcore_barrier()
    pltpu.sync_copy(den_sp.at[pl.ds(s * ROWS, ROWS)],
                    den_hbm.at[c, pl.ds(s * ROWS, ROWS)])


@functools.partial(
    pl.kernel,
    out_type=jax.ShapeDtypeStruct((NC, NP, OUT), jnp.float32),
    mesh=_mesh,
    compiler_params=_scparams,
    scratch_types=[
        pltpu.VMEM((CPT, B), jnp.int32),
        pltpu.VMEM((CPT, B), jnp.int32),
        pltpu.VMEM((CPT, B), jnp.float32),
        pltpu.VMEM((B,), jnp.float32),
        pltpu.VMEM((NP,), jnp.float32),
        pltpu.VMEM((B, OUT), jnp.float32),
        pltpu.VMEM((B, OUT), jnp.float32),
        pltpu.VMEM((B, OUT), jnp.float32),
        pltpu.VMEM_SHARED((NP, OUT), jnp.float32),
        pltpu.SemaphoreType.DMA,
        pltpu.SemaphoreType.DMA,
    ],
)
def _sc2_pass2(src_hbm, dst_hbm, ex_hbm, den_hbm, h_hbm, z_hbm, out_hbm,
               src_v, dst_v, exs_v, al_v, dn_v, h0, h1, msg_v,
               out_sp, smh0, smh1):
    c = lax.axis_index("c")
    s = lax.axis_index("s")
    wid = c * NS + s
    pltpu.sync_copy(den_hbm, dn_v)
    pltpu.sync_copy(z_hbm.at[pl.ds(s * ROWS, ROWS), :],
                    out_sp.at[pl.ds(s * ROWS, ROWS), :])
    pltpu.sync_copy(src_hbm.at[pl.ds(wid * CPT, CPT), :], src_v)
    pltpu.sync_copy(dst_hbm.at[pl.ds(wid * CPT, CPT), :], dst_v)
    pltpu.sync_copy(ex_hbm.at[pl.ds(wid * CPT, CPT), :], exs_v)
    plsc.subcore_barrier()

    bufs = ((h0, smh0), (h1, smh1))
    for par in range(2):
        h, smh = bufs[par]
        pltpu.async_copy(h_hbm.at[src_v.at[par]], h, smh)

    def pair(jj, carry):
        for par in range(2):
            j = jj * 2 + par
            h, smh = bufs[par]

            @plsc.parallel_loop(0, B // L, 1, unroll=4)
            def _(j2):
                sl = pl.ds(j2 * L, L)
                den = plsc.load_gather(dn_v, [dst_v[j, sl]])
                al_v[sl] = exs_v[j, sl] * den

            pltpu.make_async_copy(h_hbm.at[src_v.at[j]], h, smh).wait()

            @plsc.parallel_loop(0, B // L, 1, unroll=1)
            def _(g):
                avall = al_v[pl.ds(g * L, L)]
                for m in range(L):
                    av = _bcast(avall, jnp.full((L,), m, jnp.int32))
                    i = g * L + m
                    for k in range(OUT // HID):
                        msg_v[i, pl.ds(k * HID, HID)] = (
                            h[i, pl.ds(k * HID, HID)] * av)

            pltpu.sync_copy(msg_v, out_sp.at[dst_v.at[j]], add=True)

            @pl.when(j + 2 < CPT)
            def _():
                pltpu.async_copy(h_hbm.at[src_v.at[j + 2]], h, smh)

        return carry

    lax.fori_loop(0, CPT // 2, pair, 0)
    plsc.subcore_barrier()
    pltpu.sync_copy(out_sp.at[pl.ds(s * ROWS, ROWS), :],
                    out_hbm.at[c, pl.ds(s * ROWS, ROWS), :])


def kernel(x, edge_index, W1, a_src1, a_dst1, b1, W2, a_src2, a_dst2, b2):
    x_pad = jnp.pad(x, ((0, NP - N), (0, 0)))
    src = edge_index[0].astype(jnp.int32)
    dst = edge_index[1].astype(jnp.int32)
    pad = N + jnp.arange(EP - E, dtype=jnp.int32) % (NP - N)
    src_p = jnp.concatenate([src, pad]).reshape(NCHUNK_P, B)
    dst_p = jnp.concatenate([dst, pad]).reshape(NCHUNK_P, B)

    cols = jnp.arange(HEADS * HID)
    orig = (cols % HEADS) * HID + cols // HEADS
    W1p = W1[:, orig]
    b1p = b1[orig]
    W2p = W2[orig, :]

    eye = jnp.repeat(jnp.eye(HEADS, dtype=jnp.float32), HID, axis=0)
    A_src = eye * a_src1.reshape(-1)[:, None]
    A_dst = eye * a_dst1.reshape(-1)[:, None]
    A1 = jnp.concatenate([A_src, A_dst], axis=1)[orig, :]

    h1, ta1, tb1, mx1 = _tc1(x_pad, W1p, A1)
    k1 = mx1[0, :8] + mx1[0, 8:]
    k1 = jnp.maximum(k1, 0.2 * k1)
    k16_1 = jnp.concatenate([k1, k1])

    z16 = jnp.zeros((NP, 16), jnp.float32)
    z128 = jnp.zeros((NP, HEADS * HID), jnp.float32)
    den1p, ex1 = _sc1_pass1(src_p.reshape(NCHUNK_P2, B2),
                            dst_p.reshape(NCHUNK_P2, B2),
                            ta1, tb1, k16_1, z16)
    rden1 = _tc_radd(den1p[0], den1p[1])
    out1p = _sc1_pass2(src_p, dst_p, ex1, rden1, h1, z128)

    A2 = jnp.zeros((OUT, 16), jnp.float32)
    A2 = A2.at[:, 0].set(a_src2[0]).at[:, 1].set(a_dst2[0])
    h2, sa2, mx2 = _tc2(out1p[0], out1p[1], b1p.reshape(1, -1), W2p, A2)
    k2 = mx2[0, 0] + mx2[0, 1]
    k2 = jnp.maximum(k2, 0.2 * k2)
    k16_2 = jnp.full((L,), k2, jnp.float32)
    as2 = sa2[:, 0]
    ad2 = sa2[:, 1]

    z1d = jnp.zeros((NP,), jnp.float32)
    z64 = jnp.zeros((NP, OUT), jnp.float32)
    den2p, ex2 = _sc2_pass1(src_p, dst_p, as2, ad2, k16_2, z1d)
    rden2 = _tc_radd(den2p[0].reshape(80, 128),
                     den2p[1].reshape(80, 128)).reshape(NP)
    out2p = _sc2_pass2(src_p, dst_p, ex2, rden2, h2, z64)

    o = _tc3(out2p[0], out2p[1], b2.reshape(1, -1))
    return o[:N]
